# baseline (speedup 1.0000x reference)
import jax
import jax.numpy as jnp
from jax.experimental import pallas as pl


def _gatv2(x, src, dst, ea, p, n):
    xl = x @ p['Wl'] + p['bl']
    xr = x @ p['Wr'] + p['br']
    eproj = ea @ p['We']
    m = xl[src] + xr[dst] + eproj
    m = jax.nn.leaky_relu(m, negative_slope=0.2)
    alpha = jnp.sum(m * p['att'], axis=-1)
    amax = jax.ops.segment_max(alpha, dst, num_segments=n)
    alpha = jnp.exp(alpha - amax[dst])
    denom = jax.ops.segment_sum(alpha, dst, num_segments=n)
    alpha = alpha / (denom[dst] + 1e-16)
    out = jax.ops.segment_sum(xl[src] * alpha[:, None], dst, num_segments=n)
    return out + p['bias']


def kernel(x, edge_index, edge_attr, params):
    n = x.shape[0]
    loops = jnp.arange(n, dtype=edge_index.dtype)
    src = jnp.concatenate([edge_index[0], loops])
    dst = jnp.concatenate([edge_index[1], loops])
    loop_attr = jnp.broadcast_to(jnp.mean(edge_attr, axis=0, keepdims=True), (n, edge_attr.shape[1]))
    ea = jnp.concatenate([edge_attr, loop_attr], axis=0)
    h = x
    for p in params['layers']:
        h = jnp.tanh(_gatv2(h, src, dst, ea, p, n))
    h = jnp.tanh(h @ params['W_fc1'] + params['b_fc1'])
    h = jnp.tanh(h @ params['W_fc2'] + params['b_fc2'])
    return h


# speedup vs baseline: 6.8613x; 6.8613x over previous
import functools

import jax
import jax.numpy as jnp
from jax import lax
from jax.experimental import pallas as pl
from jax.experimental.pallas import tpu as pltpu
from jax.experimental.pallas import tpu_sc as plsc

N = 10000
E = 320000
D = 128
ED = 16
NP = 10240
EP = 331776
NW = 32
PERW = EP // NW
B = 64
NB = PERW // B
RPT = NP // 16


def _colsum_body(ea_ref, out_ref):
    i = pl.program_id(0)

    @pl.when(i == 0)
    def _init():
        out_ref[...] = jnp.zeros_like(out_ref)

    out_ref[0:1, :] += jnp.sum(ea_ref[...], axis=0, keepdims=True)


def _colsum(ea):
    return pl.pallas_call(
        _colsum_body,
        grid=(500,),
        in_specs=[pl.BlockSpec((E // 500, ED), lambda i: (i, 0))],
        out_specs=pl.BlockSpec((8, ED), lambda i: (0, 0)),
        out_shape=jax.ShapeDtypeStruct((8, ED), jnp.float32),
    )(ea)


def _eproj_body(ea_ref, we_ref, out_ref):
    out_ref[...] = jnp.dot(ea_ref[...], we_ref[...],
                           preferred_element_type=jnp.float32, precision=lax.Precision.HIGHEST)


def _eproj(ea_full, we):
    blk = 2048
    return pl.pallas_call(
        _eproj_body,
        grid=(EP // blk,),
        in_specs=[pl.BlockSpec((blk, ED), lambda i: (i, 0)),
                  pl.BlockSpec((ED, D), lambda i: (0, 0))],
        out_specs=pl.BlockSpec((blk, D), lambda i: (i, 0)),
        out_shape=jax.ShapeDtypeStruct((EP, D), jnp.float32),
    )(ea_full, we)


def _prep1_body(h_ref, wl_ref, bl_ref, wr_ref, br_ref, xl_ref, xr_ref):
    h = h_ref[...]
    xl_ref[...] = jnp.dot(h, wl_ref[...], preferred_element_type=jnp.float32, precision=lax.Precision.HIGHEST) + bl_ref[...]
    xr_ref[...] = jnp.dot(h, wr_ref[...], preferred_element_type=jnp.float32, precision=lax.Precision.HIGHEST) + br_ref[...]


def _prep1(h, wl, bl, wr, br):
    blk = 1024
    full = lambda shp: pl.BlockSpec(shp, lambda i: (0,) * len(shp))
    return pl.pallas_call(
        _prep1_body,
        grid=(NP // blk,),
        in_specs=[pl.BlockSpec((blk, D), lambda i: (i, 0)),
                  full((D, D)), full((1, D)), full((D, D)), full((1, D))],
        out_specs=[pl.BlockSpec((blk, D), lambda i: (i, 0)),
                   pl.BlockSpec((blk, D), lambda i: (i, 0))],
        out_shape=[jax.ShapeDtypeStruct((NP, D), jnp.float32),
                   jax.ShapeDtypeStruct((NP, D), jnp.float32)],
    )(h, wl, bl, wr, br)


def _den_col(den_blk):
    ones32 = jnp.ones((NW, 1), jnp.float32)
    return lax.dot_general(den_blk, ones32, (((0,), (0,)), ((), ())),
                           preferred_element_type=jnp.float32, precision=lax.Precision.HIGHEST)


def _prep_body(num_ref, den_ref, bias_ref, wl_ref, bl_ref, wr_ref, br_ref,
               xl_ref, xr_ref):
    num = num_ref[0] + num_ref[1]
    den = _den_col(den_ref[...])
    h = jnp.tanh(num / (den + 1e-16) + bias_ref[...])
    xl_ref[...] = jnp.dot(h, wl_ref[...], preferred_element_type=jnp.float32, precision=lax.Precision.HIGHEST) + bl_ref[...]
    xr_ref[...] = jnp.dot(h, wr_ref[...], preferred_element_type=jnp.float32, precision=lax.Precision.HIGHEST) + br_ref[...]


def _prep(num, den, bias, wl, bl, wr, br):
    blk = 1024
    full = lambda shp: pl.BlockSpec(shp, lambda i: (0,) * len(shp))
    return pl.pallas_call(
        _prep_body,
        grid=(NP // blk,),
        in_specs=[pl.BlockSpec((2, blk, D), lambda i: (0, i, 0)),
                  pl.BlockSpec((NW, blk), lambda i: (0, i)),
                  full((1, D)), full((D, D)), full((1, D)), full((D, D)), full((1, D))],
        out_specs=[pl.BlockSpec((blk, D), lambda i: (i, 0)),
                   pl.BlockSpec((blk, D), lambda i: (i, 0))],
        out_shape=[jax.ShapeDtypeStruct((NP, D), jnp.float32),
                   jax.ShapeDtypeStruct((NP, D), jnp.float32)],
    )(num, den, bias, wl, bl, wr, br)


def _final_body(num_ref, den_ref, bias_ref, w1_ref, b1_ref, w2_ref, b2_ref,
                out_ref):
    num = num_ref[0] + num_ref[1]
    den = _den_col(den_ref[...])
    h = jnp.tanh(num / (den + 1e-16) + bias_ref[...])
    z = jnp.tanh(jnp.dot(h, w1_ref[...], preferred_element_type=jnp.float32, precision=lax.Precision.HIGHEST) + b1_ref[...])
    out_ref[...] = jnp.tanh(jnp.dot(z, w2_ref[...], preferred_element_type=jnp.float32, precision=lax.Precision.HIGHEST) + b2_ref[...])


def _final(num, den, bias, w1, b1, w2, b2):
    blk = 1024
    full = lambda shp: pl.BlockSpec(shp, lambda i: (0,) * len(shp))
    outp = pl.pallas_call(
        _final_body,
        grid=(NP // blk,),
        in_specs=[pl.BlockSpec((2, blk, D), lambda i: (0, i, 0)),
                  pl.BlockSpec((NW, blk), lambda i: (0, i)),
                  full((1, D)), full((D, D)), full((1, D)), full((D, 1)), full((1, 1))],
        out_specs=pl.BlockSpec((blk, 1), lambda i: (i, 0)),
        out_shape=jax.ShapeDtypeStruct((NP, 1), jnp.float32),
    )(num, den, bias, w1, b1, w2, b2)
    return outp[:N]


def _sc_edge_body(src_hbm, dst_hbm, ep_hbm, xl_hbm, xr_hbm, att_hbm,
                  num_out, den_out,
                  idx_s, idx_d, rows_xl, rows_xr, epb,
                  abuf, attv, dent, sem, num_sh):
    c = lax.axis_index("c")
    s = lax.axis_index("s")
    wid = s * 2 + c

    pltpu.sync_copy(att_hbm, attv)

    z16 = jnp.zeros((16,), jnp.float32)

    def _zrow(r, carry):
        for k in range(8):
            rows_xl[r, pl.ds(16 * k, 16)] = z16
        return carry

    lax.fori_loop(0, B, _zrow, 0)

    def _zden(r, carry):
        dent[pl.ds(r * 16, 16)] = z16
        return carry

    lax.fori_loop(0, NP // 16, _zden, 0)

    r0 = s * RPT

    def _zslice(j, carry):
        pltpu.sync_copy(rows_xl, num_sh.at[pl.ds(r0 + j * B, B)])
        return carry

    lax.fori_loop(0, RPT // B, _zslice, 0)
    plsc.subcore_barrier()

    attc = [attv[pl.ds(16 * k, 16)] for k in range(8)]

    def _batch(b, carry):
        base = wid * PERW + b * B
        pltpu.sync_copy(src_hbm.at[pl.ds(base, B)], idx_s)
        pltpu.sync_copy(dst_hbm.at[pl.ds(base, B)], idx_d)
        cp1 = pltpu.async_copy(xl_hbm.at[idx_s], rows_xl, sem)
        cp2 = pltpu.async_copy(xr_hbm.at[idx_d], rows_xr, sem)
        cp3 = pltpu.async_copy(ep_hbm.at[pl.ds(base, B)], epb, sem)
        cp1.wait()
        cp2.wait()
        cp3.wait()

        lane = jax.lax.iota(jnp.int32, 16)

        def _grp1(g, carry1):
            reg = jnp.zeros((16,), jnp.float32)
            for l in range(16):
                e = g * 16 + l
                acc = None
                for k in range(8):
                    sl = pl.ds(16 * k, 16)
                    m = rows_xl[e, sl] + rows_xr[e, sl] + epb[e, sl]
                    m = jnp.maximum(m, 0.2 * m)
                    t = m * attc[k]
                    acc = t if acc is None else acc + t
                s = jnp.sum(acc)
                reg = jnp.where(lane == l, s, reg)
            abuf[pl.ds(g * 16, 16)] = jnp.exp(reg)
            return carry1

        lax.fori_loop(0, B // 16, _grp1, 0)

        def _grp3(g, carry3):
            av = abuf[pl.ds(g * 16, 16)]
            for l in range(16):
                e = g * 16 + l
                a = av[l]
                for k in range(8):
                    sl = pl.ds(16 * k, 16)
                    rows_xr[e, sl] = rows_xl[e, sl] * a
            i16 = idx_d[pl.ds(g * 16, 16)]
            plsc.addupdate_scatter(dent, [i16], av)
            return carry3

        lax.fori_loop(0, B // 16, _grp3, 0)

        pltpu.sync_copy(rows_xr, num_sh.at[idx_d], add=True)
        return carry

    lax.fori_loop(0, NB, _batch, 0)
    plsc.subcore_barrier()

    off = c * NP + r0
    pltpu.sync_copy(num_sh.at[pl.ds(r0, RPT)], num_out.at[pl.ds(off, RPT)])
    pltpu.sync_copy(dent, den_out.at[wid])


_sc_edge = pl.kernel(
    _sc_edge_body,
    out_type=(jax.ShapeDtypeStruct((2 * NP, D), jnp.float32),
              jax.ShapeDtypeStruct((NW, NP), jnp.float32)),
    mesh=plsc.VectorSubcoreMesh(core_axis_name="c", subcore_axis_name="s"),
    compiler_params=pltpu.CompilerParams(needs_layout_passes=False),
    scratch_types=[
        pltpu.VMEM((B,), jnp.int32),
        pltpu.VMEM((B,), jnp.int32),
        pltpu.VMEM((B, D), jnp.float32),
        pltpu.VMEM((B, D), jnp.float32),
        pltpu.VMEM((B, D), jnp.float32),
        pltpu.VMEM((B,), jnp.float32),
        pltpu.VMEM((D,), jnp.float32),
        pltpu.VMEM((NP,), jnp.float32),
        pltpu.SemaphoreType.DMA,
        pltpu.VMEM_SHARED((NP, D), jnp.float32),
    ],
)


def kernel(x, edge_index, edge_attr, params):
    colmean = (_colsum(edge_attr)[0:1] / jnp.float32(E))

    ea_full = jnp.concatenate([
        edge_attr,
        jnp.broadcast_to(colmean, (N, ED)),
        jnp.zeros((EP - E - N, ED), jnp.float32),
    ], axis=0)

    loops = jnp.arange(N, dtype=jnp.int32)
    padi = jnp.full((EP - E - N,), N, jnp.int32)
    src = jnp.concatenate([edge_index[0], loops, padi])
    dst = jnp.concatenate([edge_index[1], loops, padi])

    h = jnp.zeros((NP, D), jnp.float32).at[:N].set(x)

    layers = params['layers']
    xl, xr = _prep1(h, layers[0]['Wl'], layers[0]['bl'].reshape(1, D),
                    layers[0]['Wr'], layers[0]['br'].reshape(1, D))

    for li, p in enumerate(layers):
        epj = _eproj(ea_full, p['We'])
        num, den = _sc_edge(src, dst, epj, xl, xr, p['att'])
        num = num.reshape(2, NP, D)
        if li < 4:
            q = layers[li + 1]
            xl, xr = _prep(num, den, p['bias'].reshape(1, D),
                           q['Wl'], q['bl'].reshape(1, D),
                           q['Wr'], q['br'].reshape(1, D))
        else:
            out = _final(num, den, p['bias'].reshape(1, D),
                         params['W_fc1'], params['b_fc1'].reshape(1, D),
                         params['W_fc2'], params['b_fc2'].reshape(1, 1))
    return out


# speedup vs baseline: 6.9706x; 1.0159x over previous
import functools

import jax
import jax.numpy as jnp
from jax import lax
from jax.experimental import pallas as pl
from jax.experimental.pallas import tpu as pltpu
from jax.experimental.pallas import tpu_sc as plsc

N = 10000
E = 320000
D = 128
ED = 16
NP = 10240
EP = 331776
NW = 32
PERW = EP // NW
B = 64
NB = PERW // B
RPT = NP // 16


def _colsum_body(ea_ref, out_ref):
    i = pl.program_id(0)

    @pl.when(i == 0)
    def _init():
        out_ref[...] = jnp.zeros_like(out_ref)

    out_ref[0:1, :] += jnp.sum(ea_ref[...], axis=0, keepdims=True)


def _colsum(ea):
    return pl.pallas_call(
        _colsum_body,
        grid=(500,),
        in_specs=[pl.BlockSpec((E // 500, ED), lambda i: (i, 0))],
        out_specs=pl.BlockSpec((8, ED), lambda i: (0, 0)),
        out_shape=jax.ShapeDtypeStruct((8, ED), jnp.float32),
    )(ea)


def _eproj_body(ea_ref, we_ref, out_ref):
    out_ref[...] = jnp.dot(ea_ref[...], we_ref[...],
                           preferred_element_type=jnp.float32)


def _eproj(ea_full, we):
    blk = 2048
    return pl.pallas_call(
        _eproj_body,
        grid=(EP // blk,),
        in_specs=[pl.BlockSpec((blk, ED), lambda i: (i, 0)),
                  pl.BlockSpec((ED, D), lambda i: (0, 0))],
        out_specs=pl.BlockSpec((blk, D), lambda i: (i, 0)),
        out_shape=jax.ShapeDtypeStruct((EP, D), jnp.float32),
    )(ea_full, we)


def _prep1_body(h_ref, wl_ref, bl_ref, wr_ref, br_ref, xl_ref, xr_ref):
    h = h_ref[...]
    xl_ref[...] = jnp.dot(h, wl_ref[...], preferred_element_type=jnp.float32) + bl_ref[...]
    xr_ref[...] = jnp.dot(h, wr_ref[...], preferred_element_type=jnp.float32) + br_ref[...]


def _prep1(h, wl, bl, wr, br):
    blk = 1024
    full = lambda shp: pl.BlockSpec(shp, lambda i: (0,) * len(shp))
    return pl.pallas_call(
        _prep1_body,
        grid=(NP // blk,),
        in_specs=[pl.BlockSpec((blk, D), lambda i: (i, 0)),
                  full((D, D)), full((1, D)), full((D, D)), full((1, D))],
        out_specs=[pl.BlockSpec((blk, D), lambda i: (i, 0)),
                   pl.BlockSpec((blk, D), lambda i: (i, 0))],
        out_shape=[jax.ShapeDtypeStruct((NP, D), jnp.float32),
                   jax.ShapeDtypeStruct((NP, D), jnp.float32)],
    )(h, wl, bl, wr, br)


def _den_col(den_blk):
    ones32 = jnp.ones((NW, 1), jnp.float32)
    return lax.dot_general(den_blk, ones32, (((0,), (0,)), ((), ())),
                           preferred_element_type=jnp.float32,
                           precision=lax.Precision.HIGHEST)


def _prep_body(num_ref, den_ref, bias_ref, wl_ref, bl_ref, wr_ref, br_ref,
               xl_ref, xr_ref):
    num = num_ref[0] + num_ref[1]
    den = _den_col(den_ref[...])
    h = jnp.tanh(num / (den + 1e-16) + bias_ref[...])
    xl_ref[...] = jnp.dot(h, wl_ref[...], preferred_element_type=jnp.float32) + bl_ref[...]
    xr_ref[...] = jnp.dot(h, wr_ref[...], preferred_element_type=jnp.float32) + br_ref[...]


def _prep(num, den, bias, wl, bl, wr, br):
    blk = 1024
    full = lambda shp: pl.BlockSpec(shp, lambda i: (0,) * len(shp))
    return pl.pallas_call(
        _prep_body,
        grid=(NP // blk,),
        in_specs=[pl.BlockSpec((2, blk, D), lambda i: (0, i, 0)),
                  pl.BlockSpec((NW, blk), lambda i: (0, i)),
                  full((1, D)), full((D, D)), full((1, D)), full((D, D)), full((1, D))],
        out_specs=[pl.BlockSpec((blk, D), lambda i: (i, 0)),
                   pl.BlockSpec((blk, D), lambda i: (i, 0))],
        out_shape=[jax.ShapeDtypeStruct((NP, D), jnp.float32),
                   jax.ShapeDtypeStruct((NP, D), jnp.float32)],
    )(num, den, bias, wl, bl, wr, br)


def _final_body(num_ref, den_ref, bias_ref, w1_ref, b1_ref, w2_ref, b2_ref,
                out_ref):
    num = num_ref[0] + num_ref[1]
    den = _den_col(den_ref[...])
    h = jnp.tanh(num / (den + 1e-16) + bias_ref[...])
    z = jnp.tanh(jnp.dot(h, w1_ref[...], preferred_element_type=jnp.float32) + b1_ref[...])
    out_ref[...] = jnp.tanh(jnp.dot(z, w2_ref[...], preferred_element_type=jnp.float32) + b2_ref[...])


def _final(num, den, bias, w1, b1, w2, b2):
    blk = 1024
    full = lambda shp: pl.BlockSpec(shp, lambda i: (0,) * len(shp))
    outp = pl.pallas_call(
        _final_body,
        grid=(NP // blk,),
        in_specs=[pl.BlockSpec((2, blk, D), lambda i: (0, i, 0)),
                  pl.BlockSpec((NW, blk), lambda i: (0, i)),
                  full((1, D)), full((D, D)), full((1, D)), full((D, 1)), full((1, 1))],
        out_specs=pl.BlockSpec((blk, 1), lambda i: (i, 0)),
        out_shape=jax.ShapeDtypeStruct((NP, 1), jnp.float32),
    )(num, den, bias, w1, b1, w2, b2)
    return outp[:N]


def _sc_edge_body(src_hbm, dst_hbm, ep_hbm, xl_hbm, xr_hbm, att_hbm,
                  num_out, den_out,
                  idx_s, idx_d, rows_xl, rows_xr, epb,
                  abuf, attv, dent, sem, num_sh):
    c = lax.axis_index("c")
    s = lax.axis_index("s")
    wid = s * 2 + c

    pltpu.sync_copy(att_hbm, attv)

    z16 = jnp.zeros((16,), jnp.float32)

    def _zrow(r, carry):
        for k in range(8):
            rows_xl[r, pl.ds(16 * k, 16)] = z16
        return carry

    lax.fori_loop(0, B, _zrow, 0)

    def _zden(r, carry):
        dent[pl.ds(r * 16, 16)] = z16
        return carry

    lax.fori_loop(0, NP // 16, _zden, 0)

    r0 = s * RPT

    def _zslice(j, carry):
        pltpu.sync_copy(rows_xl, num_sh.at[pl.ds(r0 + j * B, B)])
        return carry

    lax.fori_loop(0, RPT // B, _zslice, 0)
    plsc.subcore_barrier()

    attc = [attv[pl.ds(16 * k, 16)] for k in range(8)]

    def _batch(b, carry):
        base = wid * PERW + b * B
        pltpu.sync_copy(src_hbm.at[pl.ds(base, B)], idx_s)
        pltpu.sync_copy(dst_hbm.at[pl.ds(base, B)], idx_d)
        cp1 = pltpu.async_copy(xl_hbm.at[idx_s], rows_xl, sem)
        cp2 = pltpu.async_copy(xr_hbm.at[idx_d], rows_xr, sem)
        cp3 = pltpu.async_copy(ep_hbm.at[pl.ds(base, B)], epb, sem)
        cp1.wait()
        cp2.wait()
        cp3.wait()

        lane = jax.lax.iota(jnp.int32, 16)

        def _grp1(g, carry1):
            reg = jnp.zeros((16,), jnp.float32)
            for l in range(16):
                e = g * 16 + l
                acc = None
                for k in range(8):
                    sl = pl.ds(16 * k, 16)
                    m = rows_xl[e, sl] + rows_xr[e, sl] + epb[e, sl]
                    m = jnp.maximum(m, 0.2 * m)
                    t = m * attc[k]
                    acc = t if acc is None else acc + t
                s = jnp.sum(acc)
                reg = jnp.where(lane == l, s, reg)
            abuf[pl.ds(g * 16, 16)] = jnp.exp(reg)
            return carry1

        lax.fori_loop(0, B // 16, _grp1, 0)

        def _grp3(g, carry3):
            av = abuf[pl.ds(g * 16, 16)]
            for l in range(16):
                e = g * 16 + l
                a = av[l]
                for k in range(8):
                    sl = pl.ds(16 * k, 16)
                    rows_xr[e, sl] = rows_xl[e, sl] * a
            i16 = idx_d[pl.ds(g * 16, 16)]
            plsc.addupdate_scatter(dent, [i16], av)
            return carry3

        lax.fori_loop(0, B // 16, _grp3, 0)

        pltpu.sync_copy(rows_xr, num_sh.at[idx_d], add=True)
        return carry

    lax.fori_loop(0, NB, _batch, 0)
    plsc.subcore_barrier()

    off = c * NP + r0
    pltpu.sync_copy(num_sh.at[pl.ds(r0, RPT)], num_out.at[pl.ds(off, RPT)])
    pltpu.sync_copy(dent, den_out.at[wid])


_sc_edge = pl.kernel(
    _sc_edge_body,
    out_type=(jax.ShapeDtypeStruct((2 * NP, D), jnp.float32),
              jax.ShapeDtypeStruct((NW, NP), jnp.float32)),
    mesh=plsc.VectorSubcoreMesh(core_axis_name="c", subcore_axis_name="s"),
    compiler_params=pltpu.CompilerParams(needs_layout_passes=False),
    scratch_types=[
        pltpu.VMEM((B,), jnp.int32),
        pltpu.VMEM((B,), jnp.int32),
        pltpu.VMEM((B, D), jnp.float32),
        pltpu.VMEM((B, D), jnp.float32),
        pltpu.VMEM((B, D), jnp.float32),
        pltpu.VMEM((B,), jnp.float32),
        pltpu.VMEM((D,), jnp.float32),
        pltpu.VMEM((NP,), jnp.float32),
        pltpu.SemaphoreType.DMA,
        pltpu.VMEM_SHARED((NP, D), jnp.float32),
    ],
)


def kernel(x, edge_index, edge_attr, params):
    colmean = (_colsum(edge_attr)[0:1] / jnp.float32(E))

    ea_full = jnp.concatenate([
        edge_attr,
        jnp.broadcast_to(colmean, (N, ED)),
        jnp.zeros((EP - E - N, ED), jnp.float32),
    ], axis=0)

    loops = jnp.arange(N, dtype=jnp.int32)
    padi = jnp.full((EP - E - N,), N, jnp.int32)
    src = jnp.concatenate([edge_index[0], loops, padi])
    dst = jnp.concatenate([edge_index[1], loops, padi])

    h = jnp.zeros((NP, D), jnp.float32).at[:N].set(x)

    layers = params['layers']
    xl, xr = _prep1(h, layers[0]['Wl'], layers[0]['bl'].reshape(1, D),
                    layers[0]['Wr'], layers[0]['br'].reshape(1, D))

    for li, p in enumerate(layers):
        epj = _eproj(ea_full, p['We'])
        num, den = _sc_edge(src, dst, epj, xl, xr, p['att'])
        num = num.reshape(2, NP, D)
        if li < 4:
            q = layers[li + 1]
            xl, xr = _prep(num, den, p['bias'].reshape(1, D),
                           q['Wl'], q['bl'].reshape(1, D),
                           q['Wr'], q['br'].reshape(1, D))
        else:
            out = _final(num, den, p['bias'].reshape(1, D),
                         params['W_fc1'], params['b_fc1'].reshape(1, D),
                         params['W_fc2'], params['b_fc2'].reshape(1, 1))
    return out


# speedup vs baseline: 8.6631x; 1.2428x over previous
import functools

import jax
import jax.numpy as jnp
from jax import lax
from jax.experimental import pallas as pl
from jax.experimental.pallas import tpu as pltpu
from jax.experimental.pallas import tpu_sc as plsc

N = 10000
E = 320000
D = 128
ED = 16
NP = 10240
EP = 331776
NW = 32
PERW = EP // NW
B = 32
NB = PERW // B
RPT = NP // 16


def _colsum_body(ea_ref, out_ref):
    i = pl.program_id(0)

    @pl.when(i == 0)
    def _init():
        out_ref[...] = jnp.zeros_like(out_ref)

    out_ref[0:1, :] += jnp.sum(ea_ref[...], axis=0, keepdims=True)


def _colsum(ea):
    return pl.pallas_call(
        _colsum_body,
        grid=(500,),
        in_specs=[pl.BlockSpec((E // 500, ED), lambda i: (i, 0))],
        out_specs=pl.BlockSpec((8, ED), lambda i: (0, 0)),
        out_shape=jax.ShapeDtypeStruct((8, ED), jnp.float32),
    )(ea)


def _eproj_body(ea_ref, we_ref, out_ref):
    out_ref[...] = jnp.dot(ea_ref[...], we_ref[...],
                           preferred_element_type=jnp.float32)


def _eproj(ea_full, we):
    blk = 2048
    return pl.pallas_call(
        _eproj_body,
        grid=(EP // blk,),
        in_specs=[pl.BlockSpec((blk, ED), lambda i: (i, 0)),
                  pl.BlockSpec((ED, D), lambda i: (0, 0))],
        out_specs=pl.BlockSpec((blk, D), lambda i: (i, 0)),
        out_shape=jax.ShapeDtypeStruct((EP, D), jnp.float32),
    )(ea_full, we)


def _prep1_body(h_ref, wl_ref, bl_ref, wr_ref, br_ref, xl_ref, xr_ref):
    h = h_ref[...]
    xl_ref[...] = jnp.dot(h, wl_ref[...], preferred_element_type=jnp.float32) + bl_ref[...]
    xr_ref[...] = jnp.dot(h, wr_ref[...], preferred_element_type=jnp.float32) + br_ref[...]


def _prep1(h, wl, bl, wr, br):
    blk = 1024
    full = lambda shp: pl.BlockSpec(shp, lambda i: (0,) * len(shp))
    return pl.pallas_call(
        _prep1_body,
        grid=(NP // blk,),
        in_specs=[pl.BlockSpec((blk, D), lambda i: (i, 0)),
                  full((D, D)), full((1, D)), full((D, D)), full((1, D))],
        out_specs=[pl.BlockSpec((blk, D), lambda i: (i, 0)),
                   pl.BlockSpec((blk, D), lambda i: (i, 0))],
        out_shape=[jax.ShapeDtypeStruct((NP, D), jnp.float32),
                   jax.ShapeDtypeStruct((NP, D), jnp.float32)],
    )(h, wl, bl, wr, br)


def _den_col(den_blk):
    ones32 = jnp.ones((NW, 1), jnp.float32)
    return lax.dot_general(den_blk, ones32, (((0,), (0,)), ((), ())),
                           preferred_element_type=jnp.float32,
                           precision=lax.Precision.HIGHEST)


def _prep_body(num_ref, den_ref, bias_ref, wl_ref, bl_ref, wr_ref, br_ref,
               xl_ref, xr_ref):
    num = num_ref[0] + num_ref[1]
    den = _den_col(den_ref[...])
    h = jnp.tanh(num / (den + 1e-16) + bias_ref[...])
    xl_ref[...] = jnp.dot(h, wl_ref[...], preferred_element_type=jnp.float32) + bl_ref[...]
    xr_ref[...] = jnp.dot(h, wr_ref[...], preferred_element_type=jnp.float32) + br_ref[...]


def _prep(num, den, bias, wl, bl, wr, br):
    blk = 1024
    full = lambda shp: pl.BlockSpec(shp, lambda i: (0,) * len(shp))
    return pl.pallas_call(
        _prep_body,
        grid=(NP // blk,),
        in_specs=[pl.BlockSpec((2, blk, D), lambda i: (0, i, 0)),
                  pl.BlockSpec((NW, blk), lambda i: (0, i)),
                  full((1, D)), full((D, D)), full((1, D)), full((D, D)), full((1, D))],
        out_specs=[pl.BlockSpec((blk, D), lambda i: (i, 0)),
                   pl.BlockSpec((blk, D), lambda i: (i, 0))],
        out_shape=[jax.ShapeDtypeStruct((NP, D), jnp.float32),
                   jax.ShapeDtypeStruct((NP, D), jnp.float32)],
    )(num, den, bias, wl, bl, wr, br)


def _final_body(num_ref, den_ref, bias_ref, w1_ref, b1_ref, w2_ref, b2_ref,
                out_ref):
    num = num_ref[0] + num_ref[1]
    den = _den_col(den_ref[...])
    h = jnp.tanh(num / (den + 1e-16) + bias_ref[...])
    z = jnp.tanh(jnp.dot(h, w1_ref[...], preferred_element_type=jnp.float32) + b1_ref[...])
    out_ref[...] = jnp.tanh(jnp.dot(z, w2_ref[...], preferred_element_type=jnp.float32) + b2_ref[...])


def _final(num, den, bias, w1, b1, w2, b2):
    blk = 1024
    full = lambda shp: pl.BlockSpec(shp, lambda i: (0,) * len(shp))
    outp = pl.pallas_call(
        _final_body,
        grid=(NP // blk,),
        in_specs=[pl.BlockSpec((2, blk, D), lambda i: (0, i, 0)),
                  pl.BlockSpec((NW, blk), lambda i: (0, i)),
                  full((1, D)), full((D, D)), full((1, D)), full((D, 1)), full((1, 1))],
        out_specs=pl.BlockSpec((blk, 1), lambda i: (i, 0)),
        out_shape=jax.ShapeDtypeStruct((NP, 1), jnp.float32),
    )(num, den, bias, w1, b1, w2, b2)
    return outp[:N]


def _sc_edge_body(sd_hbm, ep_hbm, xl_hbm, xr_hbm, att_hbm,
                  num_out, den_out,
                  idxsd0, idxsd1, idx_d,
                  rows_xl0, rows_xl1, rows_xr0, rows_xr1, epb0, epb1,
                  abuf, attv, dent,
                  semr0, semr1, semi0, semi1, num_sh):
    c = lax.axis_index("c")
    s = lax.axis_index("s")
    wid = s * 2 + c

    idxsd = [idxsd0, idxsd1]
    rows_xl = [rows_xl0, rows_xl1]
    rows_xr = [rows_xr0, rows_xr1]
    epb = [epb0, epb1]
    semr = [semr0, semr1]
    semi = [semi0, semi1]

    pltpu.sync_copy(att_hbm, attv)

    z16 = jnp.zeros((16,), jnp.float32)

    def _zrow(r, carry):
        for k in range(8):
            rows_xl0[r, pl.ds(16 * k, 16)] = z16
        return carry

    lax.fori_loop(0, B, _zrow, 0)

    def _zden(r, carry):
        dent[pl.ds(r * 16, 16)] = z16
        return carry

    lax.fori_loop(0, NP // 16, _zden, 0)

    r0 = s * RPT

    def _zslice(j, carry):
        pltpu.sync_copy(rows_xl0, num_sh.at[pl.ds(r0 + j * B, B)])
        return carry

    lax.fori_loop(0, RPT // B, _zslice, 0)

    attc = [attv[pl.ds(16 * k, 16)] for k in range(8)]
    lane = jax.lax.iota(jnp.int32, 16)
    ebase = wid * PERW
    sdbase = wid * NB * (2 * B)

    def _issue_rows(n, p):
        h1 = pltpu.async_copy(xl_hbm.at[idxsd[p].at[pl.ds(0, B)]],
                              rows_xl[p], semr[p])
        h2 = pltpu.async_copy(xr_hbm.at[idxsd[p].at[pl.ds(B, B)]],
                              rows_xr[p], semr[p])
        h3 = pltpu.async_copy(ep_hbm.at[pl.ds(ebase + n * B, B)],
                              epb[p], semr[p])
        return h1, h2, h3

    def _wait_rows(p):
        pltpu.make_async_copy(xl_hbm.at[idxsd[p].at[pl.ds(0, B)]],
                              rows_xl[p], semr[p]).wait()
        pltpu.make_async_copy(xr_hbm.at[idxsd[p].at[pl.ds(B, B)]],
                              rows_xr[p], semr[p]).wait()
        pltpu.make_async_copy(ep_hbm.at[pl.ds(ebase, B)], epb[p],
                              semr[p]).wait()

    pltpu.sync_copy(sd_hbm.at[pl.ds(sdbase, 2 * B)], idxsd0)
    pltpu.async_copy(sd_hbm.at[pl.ds(sdbase + 2 * B, 2 * B)], idxsd1, semi1)
    _issue_rows(0, 0)

    plsc.subcore_barrier()

    def _pair(g2, carry):
        for p in range(2):
            n = 2 * g2 + p
            q = 1 - p

            @pl.when(n + 1 < NB)
            def _issue_next():
                pltpu.make_async_copy(
                    sd_hbm.at[pl.ds(sdbase, 2 * B)], idxsd[q], semi[q]).wait()
                _issue_rows(n + 1, q)

            _wait_rows(p)

            for g in range(B // 16):
                idx_d[pl.ds(g * 16, 16)] = idxsd[p][pl.ds(B + g * 16, 16)]

            def _grp1(g, carry1):
                reg = jnp.zeros((16,), jnp.float32)
                for l in range(16):
                    e = g * 16 + l
                    acc = None
                    for k in range(8):
                        sl = pl.ds(16 * k, 16)
                        m = rows_xl[p][e, sl] + rows_xr[p][e, sl] + epb[p][e, sl]
                        m = jnp.maximum(m, 0.2 * m)
                        t = m * attc[k]
                        acc = t if acc is None else acc + t
                    sv = jnp.sum(acc)
                    reg = jnp.where(lane == l, sv, reg)
                abuf[pl.ds(g * 16, 16)] = jnp.exp(reg)
                return carry1

            lax.fori_loop(0, B // 16, _grp1, 0)

            def _grp3(g, carry3):
                av = abuf[pl.ds(g * 16, 16)]
                for l in range(16):
                    e = g * 16 + l
                    a = av[l]
                    for k in range(8):
                        sl = pl.ds(16 * k, 16)
                        rows_xr[p][e, sl] = rows_xl[p][e, sl] * a
                i16 = idx_d[pl.ds(g * 16, 16)]
                plsc.addupdate_scatter(dent, [i16], av)
                return carry3

            lax.fori_loop(0, B // 16, _grp3, 0)

            pltpu.sync_copy(rows_xr[p], num_sh.at[idx_d], add=True)

            @pl.when(n + 2 < NB)
            def _prefetch_idx():
                pltpu.async_copy(
                    sd_hbm.at[pl.ds(sdbase + (n + 2) * (2 * B), 2 * B)],
                    idxsd[p], semi[p])

        return carry

    lax.fori_loop(0, NB // 2, _pair, 0)
    plsc.subcore_barrier()

    off = c * NP + r0
    pltpu.sync_copy(num_sh.at[pl.ds(r0, RPT)], num_out.at[pl.ds(off, RPT)])
    pltpu.sync_copy(dent, den_out.at[wid])


_sc_edge = pl.kernel(
    _sc_edge_body,
    out_type=(jax.ShapeDtypeStruct((2 * NP, D), jnp.float32),
              jax.ShapeDtypeStruct((NW, NP), jnp.float32)),
    mesh=plsc.VectorSubcoreMesh(core_axis_name="c", subcore_axis_name="s"),
    compiler_params=pltpu.CompilerParams(needs_layout_passes=False),
    scratch_types=[
        pltpu.VMEM((2 * B,), jnp.int32),
        pltpu.VMEM((2 * B,), jnp.int32),
        pltpu.VMEM((B,), jnp.int32),
        pltpu.VMEM((B, D), jnp.float32),
        pltpu.VMEM((B, D), jnp.float32),
        pltpu.VMEM((B, D), jnp.float32),
        pltpu.VMEM((B, D), jnp.float32),
        pltpu.VMEM((B, D), jnp.float32),
        pltpu.VMEM((B, D), jnp.float32),
        pltpu.VMEM((B,), jnp.float32),
        pltpu.VMEM((D,), jnp.float32),
        pltpu.VMEM((NP,), jnp.float32),
        pltpu.SemaphoreType.DMA,
        pltpu.SemaphoreType.DMA,
        pltpu.SemaphoreType.DMA,
        pltpu.SemaphoreType.DMA,
        pltpu.VMEM_SHARED((NP, D), jnp.float32),
    ],
)


def kernel(x, edge_index, edge_attr, params):
    colmean = (_colsum(edge_attr)[0:1] / jnp.float32(E))

    ea_full = jnp.concatenate([
        edge_attr,
        jnp.broadcast_to(colmean, (N, ED)),
        jnp.zeros((EP - E - N, ED), jnp.float32),
    ], axis=0)

    loops = jnp.arange(N, dtype=jnp.int32)
    padi = jnp.full((EP - E - N,), N, jnp.int32)
    src = jnp.concatenate([edge_index[0], loops, padi])
    dst = jnp.concatenate([edge_index[1], loops, padi])
    sd = jnp.stack([src.reshape(NW, NB, B), dst.reshape(NW, NB, B)],
                   axis=2).reshape(EP * 2)

    h = jnp.zeros((NP, D), jnp.float32).at[:N].set(x)

    layers = params['layers']
    xl, xr = _prep1(h, layers[0]['Wl'], layers[0]['bl'].reshape(1, D),
                    layers[0]['Wr'], layers[0]['br'].reshape(1, D))

    for li, p in enumerate(layers):
        epj = _eproj(ea_full, p['We'])
        num, den = _sc_edge(sd, epj, xl, xr, p['att'])
        num = num.reshape(2, NP, D)
        if li < 4:
            q = layers[li + 1]
            xl, xr = _prep(num, den, p['bias'].reshape(1, D),
                           q['Wl'], q['bl'].reshape(1, D),
                           q['Wr'], q['br'].reshape(1, D))
        else:
            out = _final(num, den, p['bias'].reshape(1, D),
                         params['W_fc1'], params['b_fc1'].reshape(1, D),
                         params['W_fc2'], params['b_fc2'].reshape(1, 1))
    return out


# speedup vs baseline: 9.1794x; 1.0596x over previous
import functools

import jax
import jax.numpy as jnp
from jax import lax
from jax.experimental import pallas as pl
from jax.experimental.pallas import tpu as pltpu
from jax.experimental.pallas import tpu_sc as plsc

N = 10000
E = 320000
D = 128
ED = 16
NP = 10240
EP = 331776
NW = 32
PERW = EP // NW
B = 32
NB = PERW // B
RPT = NP // 16


def _colsum_body(ea_ref, out_ref):
    i = pl.program_id(0)

    @pl.when(i == 0)
    def _init():
        out_ref[...] = jnp.zeros_like(out_ref)

    out_ref[0:1, :] += jnp.sum(ea_ref[...], axis=0, keepdims=True)


def _colsum(ea):
    return pl.pallas_call(
        _colsum_body,
        grid=(500,),
        in_specs=[pl.BlockSpec((E // 500, ED), lambda i: (i, 0))],
        out_specs=pl.BlockSpec((8, ED), lambda i: (0, 0)),
        out_shape=jax.ShapeDtypeStruct((8, ED), jnp.float32),
    )(ea)


def _eproj_body(ea_ref, we_ref, out_ref):
    out_ref[...] = jnp.dot(ea_ref[...], we_ref[...],
                           preferred_element_type=jnp.float32)


def _eproj(ea_full, we):
    blk = 2048
    return pl.pallas_call(
        _eproj_body,
        grid=(EP // blk,),
        in_specs=[pl.BlockSpec((blk, ED), lambda i: (i, 0)),
                  pl.BlockSpec((ED, D), lambda i: (0, 0))],
        out_specs=pl.BlockSpec((blk, D), lambda i: (i, 0)),
        out_shape=jax.ShapeDtypeStruct((EP, D), jnp.float32),
    )(ea_full, we)


def _prep1_body(h_ref, wl_ref, bl_ref, wr_ref, br_ref, xl_ref, xr_ref):
    h = h_ref[...]
    xl_ref[...] = jnp.dot(h, wl_ref[...], preferred_element_type=jnp.float32) + bl_ref[...]
    xr_ref[...] = jnp.dot(h, wr_ref[...], preferred_element_type=jnp.float32) + br_ref[...]


def _prep1(h, wl, bl, wr, br):
    blk = 1024
    full = lambda shp: pl.BlockSpec(shp, lambda i: (0,) * len(shp))
    return pl.pallas_call(
        _prep1_body,
        grid=(NP // blk,),
        in_specs=[pl.BlockSpec((blk, D), lambda i: (i, 0)),
                  full((D, D)), full((1, D)), full((D, D)), full((1, D))],
        out_specs=[pl.BlockSpec((blk, D), lambda i: (i, 0)),
                   pl.BlockSpec((blk, D), lambda i: (i, 0))],
        out_shape=[jax.ShapeDtypeStruct((NP, D), jnp.float32),
                   jax.ShapeDtypeStruct((NP, D), jnp.float32)],
    )(h, wl, bl, wr, br)


def _den_col(den_blk):
    ones32 = jnp.ones((NW, 1), jnp.float32)
    return lax.dot_general(den_blk, ones32, (((0,), (0,)), ((), ())),
                           preferred_element_type=jnp.float32,
                           precision=lax.Precision.HIGHEST)


def _prep_body(num_ref, den_ref, bias_ref, wl_ref, bl_ref, wr_ref, br_ref,
               xl_ref, xr_ref):
    num = num_ref[0] + num_ref[1]
    den = _den_col(den_ref[...])
    h = jnp.tanh(num / (den + 1e-16) + bias_ref[...])
    xl_ref[...] = jnp.dot(h, wl_ref[...], preferred_element_type=jnp.float32) + bl_ref[...]
    xr_ref[...] = jnp.dot(h, wr_ref[...], preferred_element_type=jnp.float32) + br_ref[...]


def _prep(num, den, bias, wl, bl, wr, br):
    blk = 1024
    full = lambda shp: pl.BlockSpec(shp, lambda i: (0,) * len(shp))
    return pl.pallas_call(
        _prep_body,
        grid=(NP // blk,),
        in_specs=[pl.BlockSpec((2, blk, D), lambda i: (0, i, 0)),
                  pl.BlockSpec((NW, blk), lambda i: (0, i)),
                  full((1, D)), full((D, D)), full((1, D)), full((D, D)), full((1, D))],
        out_specs=[pl.BlockSpec((blk, D), lambda i: (i, 0)),
                   pl.BlockSpec((blk, D), lambda i: (i, 0))],
        out_shape=[jax.ShapeDtypeStruct((NP, D), jnp.float32),
                   jax.ShapeDtypeStruct((NP, D), jnp.float32)],
    )(num, den, bias, wl, bl, wr, br)


def _final_body(num_ref, den_ref, bias_ref, w1_ref, b1_ref, w2_ref, b2_ref,
                out_ref):
    num = num_ref[0] + num_ref[1]
    den = _den_col(den_ref[...])
    h = jnp.tanh(num / (den + 1e-16) + bias_ref[...])
    z = jnp.tanh(jnp.dot(h, w1_ref[...], preferred_element_type=jnp.float32) + b1_ref[...])
    out_ref[...] = jnp.tanh(jnp.dot(z, w2_ref[...], preferred_element_type=jnp.float32) + b2_ref[...])


def _final(num, den, bias, w1, b1, w2, b2):
    blk = 1024
    full = lambda shp: pl.BlockSpec(shp, lambda i: (0,) * len(shp))
    outp = pl.pallas_call(
        _final_body,
        grid=(NP // blk,),
        in_specs=[pl.BlockSpec((2, blk, D), lambda i: (0, i, 0)),
                  pl.BlockSpec((NW, blk), lambda i: (0, i)),
                  full((1, D)), full((D, D)), full((1, D)), full((D, 1)), full((1, 1))],
        out_specs=pl.BlockSpec((blk, 1), lambda i: (i, 0)),
        out_shape=jax.ShapeDtypeStruct((NP, 1), jnp.float32),
    )(num, den, bias, w1, b1, w2, b2)
    return outp[:N]


def _sc_edge_body(sd_hbm, ep_hbm, xl_hbm, xr_hbm, att_hbm,
                  num_out, den_out,
                  idxsd0, idxsd1, idx_d0, idx_d1,
                  rows_xl0, rows_xl1, rows_xr0, rows_xr1, epb0, epb1,
                  wrow0, wrow1, abuf, attv, dent,
                  semr0, semr1, semi0, semi1, semw0, semw1, num_sh):
    c = lax.axis_index("c")
    s = lax.axis_index("s")
    wid = s * 2 + c

    idxsd = [idxsd0, idxsd1]
    idx_d = [idx_d0, idx_d1]
    rows_xl = [rows_xl0, rows_xl1]
    rows_xr = [rows_xr0, rows_xr1]
    epb = [epb0, epb1]
    wrow = [wrow0, wrow1]
    semr = [semr0, semr1]
    semi = [semi0, semi1]
    semw = [semw0, semw1]

    pltpu.sync_copy(att_hbm, attv)

    z16 = jnp.zeros((16,), jnp.float32)

    def _zrow(r, carry):
        for k in range(8):
            rows_xl0[r, pl.ds(16 * k, 16)] = z16
        return carry

    lax.fori_loop(0, B, _zrow, 0)

    def _zden(r, carry):
        dent[pl.ds(r * 16, 16)] = z16
        return carry

    lax.fori_loop(0, NP // 16, _zden, 0)

    r0 = s * RPT

    def _zslice(j, carry):
        pltpu.sync_copy(rows_xl0, num_sh.at[pl.ds(r0 + j * B, B)])
        return carry

    lax.fori_loop(0, RPT // B, _zslice, 0)

    attc = [attv[pl.ds(16 * k, 16)] for k in range(8)]
    lane = jax.lax.iota(jnp.int32, 16)
    ebase = wid * PERW
    sdbase = wid * NB * (2 * B)

    def _issue_rows(n, p):
        h1 = pltpu.async_copy(xl_hbm.at[idxsd[p].at[pl.ds(0, B)]],
                              rows_xl[p], semr[p])
        h2 = pltpu.async_copy(xr_hbm.at[idxsd[p].at[pl.ds(B, B)]],
                              rows_xr[p], semr[p])
        h3 = pltpu.async_copy(ep_hbm.at[pl.ds(ebase + n * B, B)],
                              epb[p], semr[p])
        return h1, h2, h3

    def _wait_rows(p):
        pltpu.make_async_copy(xl_hbm.at[idxsd[p].at[pl.ds(0, B)]],
                              rows_xl[p], semr[p]).wait()
        pltpu.make_async_copy(xr_hbm.at[idxsd[p].at[pl.ds(B, B)]],
                              rows_xr[p], semr[p]).wait()
        pltpu.make_async_copy(ep_hbm.at[pl.ds(ebase, B)], epb[p],
                              semr[p]).wait()

    pltpu.sync_copy(sd_hbm.at[pl.ds(sdbase, 2 * B)], idxsd0)
    pltpu.async_copy(sd_hbm.at[pl.ds(sdbase + 2 * B, 2 * B)], idxsd1, semi1)
    _issue_rows(0, 0)

    plsc.subcore_barrier()

    def _drain_scatter(p):
        pltpu.make_async_copy(ep_hbm.at[pl.ds(0, B)], wrow[p], semw[p]).wait()

    def _pair(g2, carry):
        for p in range(2):
            n = 2 * g2 + p
            q = 1 - p

            @pl.when(n + 1 < NB)
            def _issue_next():
                pltpu.make_async_copy(
                    sd_hbm.at[pl.ds(sdbase, 2 * B)], idxsd[q], semi[q]).wait()
                _issue_rows(n + 1, q)

            _wait_rows(p)

            @pl.when(n >= 2)
            def _drain_prev():
                _drain_scatter(p)

            for g in range(B // 16):
                idx_d[p][pl.ds(g * 16, 16)] = idxsd[p][pl.ds(B + g * 16, 16)]

            def _grp1(g, carry1):
                reg = jnp.zeros((16,), jnp.float32)
                for l in range(16):
                    e = g * 16 + l
                    acc = None
                    for k in range(8):
                        sl = pl.ds(16 * k, 16)
                        m = rows_xl[p][e, sl] + rows_xr[p][e, sl] + epb[p][e, sl]
                        m = jnp.maximum(m, 0.2 * m)
                        t = m * attc[k]
                        acc = t if acc is None else acc + t
                    sv = jnp.sum(acc)
                    reg = jnp.where(lane == l, sv, reg)
                abuf[pl.ds(g * 16, 16)] = jnp.exp(reg)
                return carry1

            lax.fori_loop(0, B // 16, _grp1, 0)

            def _grp3(g, carry3):
                av = abuf[pl.ds(g * 16, 16)]
                for l in range(16):
                    e = g * 16 + l
                    a = av[l]
                    for k in range(8):
                        sl = pl.ds(16 * k, 16)
                        wrow[p][e, sl] = rows_xl[p][e, sl] * a
                i16 = idx_d[p][pl.ds(g * 16, 16)]
                plsc.addupdate_scatter(dent, [i16], av)
                return carry3

            lax.fori_loop(0, B // 16, _grp3, 0)

            pltpu.async_copy(wrow[p], num_sh.at[idx_d[p]], semw[p], add=True)

            @pl.when(n + 2 < NB)
            def _prefetch_idx():
                pltpu.async_copy(
                    sd_hbm.at[pl.ds(sdbase + (n + 2) * (2 * B), 2 * B)],
                    idxsd[p], semi[p])

        return carry

    lax.fori_loop(0, NB // 2, _pair, 0)
    _drain_scatter(0)
    _drain_scatter(1)
    plsc.subcore_barrier()

    off = c * NP + r0
    pltpu.sync_copy(num_sh.at[pl.ds(r0, RPT)], num_out.at[pl.ds(off, RPT)])
    pltpu.sync_copy(dent, den_out.at[wid])


_sc_edge = pl.kernel(
    _sc_edge_body,
    out_type=(jax.ShapeDtypeStruct((2 * NP, D), jnp.float32),
              jax.ShapeDtypeStruct((NW, NP), jnp.float32)),
    mesh=plsc.VectorSubcoreMesh(core_axis_name="c", subcore_axis_name="s"),
    compiler_params=pltpu.CompilerParams(needs_layout_passes=False),
    scratch_types=[
        pltpu.VMEM((2 * B,), jnp.int32),
        pltpu.VMEM((2 * B,), jnp.int32),
        pltpu.VMEM((B,), jnp.int32),
        pltpu.VMEM((B,), jnp.int32),
        pltpu.VMEM((B, D), jnp.float32),
        pltpu.VMEM((B, D), jnp.float32),
        pltpu.VMEM((B, D), jnp.float32),
        pltpu.VMEM((B, D), jnp.float32),
        pltpu.VMEM((B, D), jnp.float32),
        pltpu.VMEM((B, D), jnp.float32),
        pltpu.VMEM((B, D), jnp.float32),
        pltpu.VMEM((B, D), jnp.float32),
        pltpu.VMEM((B,), jnp.float32),
        pltpu.VMEM((D,), jnp.float32),
        pltpu.VMEM((NP,), jnp.float32),
        pltpu.SemaphoreType.DMA,
        pltpu.SemaphoreType.DMA,
        pltpu.SemaphoreType.DMA,
        pltpu.SemaphoreType.DMA,
        pltpu.SemaphoreType.DMA,
        pltpu.SemaphoreType.DMA,
        pltpu.VMEM_SHARED((NP, D), jnp.float32),
    ],
)


def kernel(x, edge_index, edge_attr, params):
    colmean = (_colsum(edge_attr)[0:1] / jnp.float32(E))

    ea_full = jnp.concatenate([
        edge_attr,
        jnp.broadcast_to(colmean, (N, ED)),
        jnp.zeros((EP - E - N, ED), jnp.float32),
    ], axis=0)

    loops = jnp.arange(N, dtype=jnp.int32)
    padi = jnp.full((EP - E - N,), N, jnp.int32)
    src = jnp.concatenate([edge_index[0], loops, padi])
    dst = jnp.concatenate([edge_index[1], loops, padi])
    sd = jnp.stack([src.reshape(NW, NB, B), dst.reshape(NW, NB, B)],
                   axis=2).reshape(EP * 2)

    h = jnp.zeros((NP, D), jnp.float32).at[:N].set(x)

    layers = params['layers']
    xl, xr = _prep1(h, layers[0]['Wl'], layers[0]['bl'].reshape(1, D),
                    layers[0]['Wr'], layers[0]['br'].reshape(1, D))

    for li, p in enumerate(layers):
        epj = _eproj(ea_full, p['We'])
        num, den = _sc_edge(sd, epj, xl, xr, p['att'])
        num = num.reshape(2, NP, D)
        if li < 4:
            q = layers[li + 1]
            xl, xr = _prep(num, den, p['bias'].reshape(1, D),
                           q['Wl'], q['bl'].reshape(1, D),
                           q['Wr'], q['br'].reshape(1, D))
        else:
            out = _final(num, den, p['bias'].reshape(1, D),
                         params['W_fc1'], params['b_fc1'].reshape(1, D),
                         params['W_fc2'], params['b_fc2'].reshape(1, 1))
    return out
